# Initial kernel scaffold; baseline (speedup 1.0000x reference)
import jax
import jax.numpy as jnp
from jax.experimental import pallas as pl


def kernel(x, table):
    raise NotImplementedError("write your pallas kernel here")


# speedup vs baseline: 1.8725x; 1.8725x over previous
import functools

import jax
import jax.numpy as jnp
from jax import lax
from jax.experimental import pallas as pl
from jax.experimental.pallas import tpu as pltpu
from jax.experimental.pallas import tpu_sc as plsc

VOCAB = 1000000
EMB = 64
B = 16384
L = 50

NC = 2
NS = 16
NW = NC * NS

TOTAL = B * L
PER_W = TOTAL // NW
CHUNK = 128
NCHUNK = PER_W // CHUNK
NBUF = 4


def _embed_lookup(x_grp, table):
  mesh = plsc.VectorSubcoreMesh(core_axis_name="c", subcore_axis_name="s")

  @functools.partial(
      pl.kernel,
      out_type=jax.ShapeDtypeStruct((TOTAL, EMB), jnp.float32),
      mesh=mesh,
      scratch_types=(
          [pltpu.VMEM((NCHUNK, CHUNK), jnp.int32)]
          + [pltpu.VMEM((CHUNK, EMB), jnp.float32) for _ in range(NBUF)]
          + [pltpu.SemaphoreType.DMA for _ in range(2 * NBUF)]
      ),
      compiler_params=pltpu.CompilerParams(use_tc_tiling_on_sc=False),
  )
  def k(x_hbm, table_hbm, out_hbm, idx_v, *rest):
    bufs = rest[:NBUF]
    gsems = rest[NBUF:2 * NBUF]
    ssems = rest[2 * NBUF:]

    wid = lax.axis_index("s") * NC + lax.axis_index("c")
    base = wid * PER_W

    pltpu.sync_copy(x_hbm.at[wid], idx_v)

    def start_gather(jj, b):
      pltpu.async_copy(table_hbm.at[idx_v.at[jj]], bufs[b], gsems[b])

    def wait_gather(b):
      pltpu.make_async_copy(table_hbm.at[idx_v.at[0]], bufs[b], gsems[b]).wait()

    def start_scatter(jj, b):
      dst = out_hbm.at[pl.ds(base + jj * CHUNK, CHUNK)]
      pltpu.async_copy(bufs[b], dst, ssems[b])

    def wait_scatter(b):
      dst = out_hbm.at[pl.ds(base, CHUNK)]
      pltpu.make_async_copy(bufs[b], dst, ssems[b]).wait()

    for b in range(NBUF):
      start_gather(b, b)

    @pl.loop(0, NCHUNK, step=NBUF)
    def _round(j0):
      for b in range(NBUF):
        wait_gather(b)
        start_scatter(j0 + b, b)
      for b in range(NBUF):
        jn = j0 + b + NBUF

        @pl.when(jn < NCHUNK)
        def _():
          wait_scatter(b)
          start_gather(jn, b)

    for b in range(NBUF):
      wait_scatter(b)

  return k(x_grp, table)


@jax.jit
def kernel(x, table):
  x_grp = jnp.reshape(x.astype(jnp.int32), (NW, NCHUNK, CHUNK))
  out = _embed_lookup(x_grp, table)
  return jnp.reshape(out, (B, L, EMB))
